# Initial kernel scaffold; baseline (speedup 1.0000x reference)
import functools

import jax
import jax.numpy as jnp
from jax.experimental import pallas as pl
from jax.experimental.pallas import tpu as pltpu

B = 128


def _grouped_ffn_body(te_ref, x_ref, wg_ref, wu_ref, wd_ref, wrow_ref, o_ref):
    x = x_ref[...]
    g = jnp.dot(x, wg_ref[0], preferred_element_type=jnp.float32)
    u = jnp.dot(x, wu_ref[0], preferred_element_type=jnp.float32)
    h = (g * jax.nn.sigmoid(g)) * u
    y = jnp.dot(h, wd_ref[0], preferred_element_type=jnp.float32)
    o_ref[...] = y * wrow_ref[...]


def _shared_ffn_body(x_ref, wg_ref, wu_ref, wd_ref, g0_ref, g1_ref, o_ref):
    x = x_ref[...]
    g = jnp.dot(x, wg_ref[...], preferred_element_type=jnp.float32)
    u = jnp.dot(x, wu_ref[...], preferred_element_type=jnp.float32)
    h = (g * jax.nn.sigmoid(g)) * u
    y = jnp.dot(h, wd_ref[...], preferred_element_type=jnp.float32)
    o_ref[...] = y + g0_ref[...] + g1_ref[...]


def kernel(hidden_states, topk_indices, topk_weights, w_gate, w_up, w_down,
           shared_w_gate, shared_w_up, shared_w_down):
    T, D = hidden_states.shape
    K = topk_indices.shape[1]
    E, _, F = w_gate.shape
    TK = T * K
    NT = TK // B + E
    R = NT * B

    flat_idx = topk_indices.reshape(-1).astype(jnp.int32)
    flat_w = topk_weights.reshape(-1)
    sort_ord = jnp.argsort(flat_idx, stable=True)
    sorted_e = flat_idx[sort_ord]
    counts = jnp.bincount(flat_idx, length=E)
    tiles_per_e = (counts + B - 1) // B
    pad_start = jnp.concatenate([jnp.zeros((1,), jnp.int32),
                                 jnp.cumsum(tiles_per_e * B).astype(jnp.int32)])[:E]
    cum_c = jnp.concatenate([jnp.zeros((1,), jnp.int32),
                             jnp.cumsum(counts).astype(jnp.int32)])[:E]
    j = jnp.arange(TK, dtype=jnp.int32)
    dest = pad_start[sorted_e] + j - cum_c[sorted_e]
    src_tok = jnp.zeros((R,), jnp.int32).at[dest].set(sort_ord.astype(jnp.int32) // K)
    row_w = jnp.zeros((R,), jnp.float32).at[dest].set(flat_w[sort_ord])
    comb_pos = jnp.zeros((TK,), jnp.int32).at[sort_ord].set(dest)
    tile_expert = jnp.minimum(
        jnp.searchsorted(jnp.cumsum(tiles_per_e), jnp.arange(NT), side='right'),
        E - 1).astype(jnp.int32)

    x_sorted = hidden_states[src_tok]

    grid_spec = pltpu.PrefetchScalarGridSpec(
        num_scalar_prefetch=1,
        grid=(NT,),
        in_specs=[
            pl.BlockSpec((B, D), lambda i, te: (i, 0)),
            pl.BlockSpec((1, D, F), lambda i, te: (te[i], 0, 0)),
            pl.BlockSpec((1, D, F), lambda i, te: (te[i], 0, 0)),
            pl.BlockSpec((1, F, D), lambda i, te: (te[i], 0, 0)),
            pl.BlockSpec((B, 1), lambda i, te: (i, 0)),
        ],
        out_specs=pl.BlockSpec((B, D), lambda i, te: (i, 0)),
    )
    y_routed = pl.pallas_call(
        _grouped_ffn_body,
        grid_spec=grid_spec,
        out_shape=jax.ShapeDtypeStruct((R, D), jnp.float32),
    )(tile_expert, x_sorted, w_gate, w_up, w_down, row_w[:, None])

    pos = comb_pos.reshape(T, K)
    g0 = y_routed[pos[:, 0]]
    g1 = y_routed[pos[:, 1]]

    NTS = T // B
    out = pl.pallas_call(
        _shared_ffn_body,
        grid=(NTS,),
        in_specs=[
            pl.BlockSpec((B, D), lambda i: (i, 0)),
            pl.BlockSpec((D, F), lambda i: (0, 0)),
            pl.BlockSpec((D, F), lambda i: (0, 0)),
            pl.BlockSpec((F, D), lambda i: (0, 0)),
            pl.BlockSpec((B, D), lambda i: (i, 0)),
            pl.BlockSpec((B, D), lambda i: (i, 0)),
        ],
        out_specs=pl.BlockSpec((B, D), lambda i: (i, 0)),
        out_shape=jax.ShapeDtypeStruct((T, D), jnp.float32),
    )(hidden_states, shared_w_gate, shared_w_up, shared_w_down, g0, g1)
    return out


# speedup vs baseline: 3.1879x; 3.1879x over previous
import functools

import jax
import jax.numpy as jnp
from jax.experimental import pallas as pl
from jax.experimental.pallas import tpu as pltpu

B = 128


def _gate_up_body(te_ref, x_ref, wg_ref, wu_ref, h_ref):
    x = x_ref[...]
    g = jnp.dot(x, wg_ref[0], preferred_element_type=jnp.float32)
    u = jnp.dot(x, wu_ref[0], preferred_element_type=jnp.float32)
    h_ref[...] = (g * jax.nn.sigmoid(g)) * u


def _down_body(te_ref, h_ref, wd_ref, wrow_ref, o_ref):
    y = jnp.dot(h_ref[...], wd_ref[0], preferred_element_type=jnp.float32)
    o_ref[...] = y * wrow_ref[...]


def _shared_gate_up_body(x_ref, wg_ref, wu_ref, h_ref):
    x = x_ref[...]
    g = jnp.dot(x, wg_ref[...], preferred_element_type=jnp.float32)
    u = jnp.dot(x, wu_ref[...], preferred_element_type=jnp.float32)
    h_ref[...] = (g * jax.nn.sigmoid(g)) * u


def _shared_down_body(h_ref, wd_ref, g0_ref, g1_ref, o_ref):
    y = jnp.dot(h_ref[...], wd_ref[...], preferred_element_type=jnp.float32)
    o_ref[...] = y + g0_ref[...] + g1_ref[...]


def kernel(hidden_states, topk_indices, topk_weights, w_gate, w_up, w_down,
           shared_w_gate, shared_w_up, shared_w_down):
    T, D = hidden_states.shape
    K = topk_indices.shape[1]
    E, _, F = w_gate.shape
    TK = T * K
    NT = TK // B + E
    R = NT * B

    flat_idx = topk_indices.reshape(-1).astype(jnp.int32)
    flat_w = topk_weights.reshape(-1)
    sort_ord = jnp.argsort(flat_idx, stable=True)
    sorted_e = flat_idx[sort_ord]
    counts = jnp.bincount(flat_idx, length=E)
    tiles_per_e = (counts + B - 1) // B
    pad_start = jnp.concatenate([jnp.zeros((1,), jnp.int32),
                                 jnp.cumsum(tiles_per_e * B).astype(jnp.int32)])[:E]
    cum_c = jnp.concatenate([jnp.zeros((1,), jnp.int32),
                             jnp.cumsum(counts).astype(jnp.int32)])[:E]
    j = jnp.arange(TK, dtype=jnp.int32)
    dest = pad_start[sorted_e] + j - cum_c[sorted_e]
    src_tok = jnp.zeros((R,), jnp.int32).at[dest].set(sort_ord.astype(jnp.int32) // K)
    row_w = jnp.zeros((R,), jnp.float32).at[dest].set(flat_w[sort_ord])
    comb_pos = jnp.zeros((TK,), jnp.int32).at[sort_ord].set(dest)
    tile_expert = jnp.minimum(
        jnp.searchsorted(jnp.cumsum(tiles_per_e), jnp.arange(NT), side='right'),
        E - 1).astype(jnp.int32)

    x_sorted = hidden_states[src_tok]

    h_routed = pl.pallas_call(
        _gate_up_body,
        grid_spec=pltpu.PrefetchScalarGridSpec(
            num_scalar_prefetch=1,
            grid=(NT,),
            in_specs=[
                pl.BlockSpec((B, D), lambda i, te: (i, 0)),
                pl.BlockSpec((1, D, F), lambda i, te: (te[i], 0, 0)),
                pl.BlockSpec((1, D, F), lambda i, te: (te[i], 0, 0)),
            ],
            out_specs=pl.BlockSpec((B, F), lambda i, te: (i, 0)),
        ),
        out_shape=jax.ShapeDtypeStruct((R, F), jnp.float32),
    )(tile_expert, x_sorted, w_gate, w_up)

    y_routed = pl.pallas_call(
        _down_body,
        grid_spec=pltpu.PrefetchScalarGridSpec(
            num_scalar_prefetch=1,
            grid=(NT,),
            in_specs=[
                pl.BlockSpec((B, F), lambda i, te: (i, 0)),
                pl.BlockSpec((1, F, D), lambda i, te: (te[i], 0, 0)),
                pl.BlockSpec((B, 1), lambda i, te: (i, 0)),
            ],
            out_specs=pl.BlockSpec((B, D), lambda i, te: (i, 0)),
        ),
        out_shape=jax.ShapeDtypeStruct((R, D), jnp.float32),
    )(tile_expert, h_routed, w_down, row_w[:, None])

    pos = comb_pos.reshape(T, K)
    g0 = y_routed[pos[:, 0]]
    g1 = y_routed[pos[:, 1]]

    NTS = T // B
    h_sh = pl.pallas_call(
        _shared_gate_up_body,
        grid=(NTS,),
        in_specs=[
            pl.BlockSpec((B, D), lambda i: (i, 0)),
            pl.BlockSpec((D, F), lambda i: (0, 0)),
            pl.BlockSpec((D, F), lambda i: (0, 0)),
        ],
        out_specs=pl.BlockSpec((B, F), lambda i: (i, 0)),
        out_shape=jax.ShapeDtypeStruct((T, F), jnp.float32),
    )(hidden_states, shared_w_gate, shared_w_up)

    out = pl.pallas_call(
        _shared_down_body,
        grid=(NTS,),
        in_specs=[
            pl.BlockSpec((B, F), lambda i: (i, 0)),
            pl.BlockSpec((F, D), lambda i: (0, 0)),
            pl.BlockSpec((B, D), lambda i: (i, 0)),
            pl.BlockSpec((B, D), lambda i: (i, 0)),
        ],
        out_specs=pl.BlockSpec((B, D), lambda i: (i, 0)),
        out_shape=jax.ShapeDtypeStruct((T, D), jnp.float32),
    )(h_sh, shared_w_down, g0, g1)
    return out


# speedup vs baseline: 3.1891x; 1.0004x over previous
import functools

import jax
import jax.numpy as jnp
from jax import lax
from jax.experimental import pallas as pl
from jax.experimental.pallas import tpu as pltpu
from jax.experimental.pallas import tpu_sc as plsc

B = 128

_NC, _NS = 2, 16
_NW = _NC * _NS


def _sc_row_gather(table, idx, n_rows):
    D = table.shape[1]
    rpw = n_rows // _NW
    ch = 32 if rpw % 32 == 0 else 16
    n_ch = rpw // ch
    mesh = plsc.VectorSubcoreMesh(core_axis_name="c", subcore_axis_name="s")

    @functools.partial(
        pl.kernel,
        out_type=jax.ShapeDtypeStruct((n_rows, D), jnp.float32),
        mesh=mesh,
        scratch_types=[
            pltpu.VMEM((rpw,), jnp.int32),
            pltpu.VMEM((ch, D), jnp.float32),
            pltpu.SemaphoreType.DMA,
        ],
    )
    def gather_k(table_hbm, idx_hbm, out_hbm, idx_v, rows_v, sem):
        wid = lax.axis_index("s") * _NC + lax.axis_index("c")
        base = wid * rpw
        pltpu.sync_copy(idx_hbm.at[pl.ds(base, rpw)], idx_v)

        def body(ci, carry):
            pltpu.async_copy(
                table_hbm.at[idx_v.at[pl.ds(ci * ch, ch)]], rows_v, sem
            ).wait()
            pltpu.sync_copy(rows_v, out_hbm.at[pl.ds(base + ci * ch, ch)])
            return carry

        lax.fori_loop(0, n_ch, body, 0)

    return gather_k(table, idx)


def _gate_up_body(te_ref, x_ref, wg_ref, wu_ref, h_ref):
    x = x_ref[...]
    g = jnp.dot(x, wg_ref[0], preferred_element_type=jnp.float32)
    u = jnp.dot(x, wu_ref[0], preferred_element_type=jnp.float32)
    h_ref[...] = (g * jax.nn.sigmoid(g)) * u


def _down_body(te_ref, h_ref, wd_ref, wrow_ref, o_ref):
    y = jnp.dot(h_ref[...], wd_ref[0], preferred_element_type=jnp.float32)
    o_ref[...] = y * wrow_ref[...]


def _shared_gate_up_body(x_ref, wg_ref, wu_ref, h_ref):
    x = x_ref[...]
    g = jnp.dot(x, wg_ref[...], preferred_element_type=jnp.float32)
    u = jnp.dot(x, wu_ref[...], preferred_element_type=jnp.float32)
    h_ref[...] = (g * jax.nn.sigmoid(g)) * u


def _shared_down_body(h_ref, wd_ref, g0_ref, g1_ref, o_ref):
    y = jnp.dot(h_ref[...], wd_ref[...], preferred_element_type=jnp.float32)
    o_ref[...] = y + g0_ref[...] + g1_ref[...]


def kernel(hidden_states, topk_indices, topk_weights, w_gate, w_up, w_down,
           shared_w_gate, shared_w_up, shared_w_down):
    T, D = hidden_states.shape
    K = topk_indices.shape[1]
    E, _, F = w_gate.shape
    TK = T * K
    NT = TK // B + E
    R = NT * B

    flat_idx = topk_indices.reshape(-1).astype(jnp.int32)
    flat_w = topk_weights.reshape(-1)
    sort_ord = jnp.argsort(flat_idx, stable=True)
    sorted_e = flat_idx[sort_ord]
    counts = jnp.bincount(flat_idx, length=E)
    tiles_per_e = (counts + B - 1) // B
    pad_start = jnp.concatenate([jnp.zeros((1,), jnp.int32),
                                 jnp.cumsum(tiles_per_e * B).astype(jnp.int32)])[:E]
    cum_c = jnp.concatenate([jnp.zeros((1,), jnp.int32),
                             jnp.cumsum(counts).astype(jnp.int32)])[:E]
    j = jnp.arange(TK, dtype=jnp.int32)
    dest = pad_start[sorted_e] + j - cum_c[sorted_e]
    src_tok = jnp.zeros((R,), jnp.int32).at[dest].set(sort_ord.astype(jnp.int32) // K)
    row_w = jnp.zeros((R,), jnp.float32).at[dest].set(flat_w[sort_ord])
    comb_pos = jnp.zeros((TK,), jnp.int32).at[sort_ord].set(dest)
    tile_expert = jnp.minimum(
        jnp.searchsorted(jnp.cumsum(tiles_per_e), jnp.arange(NT), side='right'),
        E - 1).astype(jnp.int32)

    x_sorted = _sc_row_gather(hidden_states, src_tok, R)

    h_routed = pl.pallas_call(
        _gate_up_body,
        grid_spec=pltpu.PrefetchScalarGridSpec(
            num_scalar_prefetch=1,
            grid=(NT,),
            in_specs=[
                pl.BlockSpec((B, D), lambda i, te: (i, 0)),
                pl.BlockSpec((1, D, F), lambda i, te: (te[i], 0, 0)),
                pl.BlockSpec((1, D, F), lambda i, te: (te[i], 0, 0)),
            ],
            out_specs=pl.BlockSpec((B, F), lambda i, te: (i, 0)),
        ),
        out_shape=jax.ShapeDtypeStruct((R, F), jnp.float32),
    )(tile_expert, x_sorted, w_gate, w_up)

    y_routed = pl.pallas_call(
        _down_body,
        grid_spec=pltpu.PrefetchScalarGridSpec(
            num_scalar_prefetch=1,
            grid=(NT,),
            in_specs=[
                pl.BlockSpec((B, F), lambda i, te: (i, 0)),
                pl.BlockSpec((1, F, D), lambda i, te: (te[i], 0, 0)),
                pl.BlockSpec((B, 1), lambda i, te: (i, 0)),
            ],
            out_specs=pl.BlockSpec((B, D), lambda i, te: (i, 0)),
        ),
        out_shape=jax.ShapeDtypeStruct((R, D), jnp.float32),
    )(tile_expert, h_routed, w_down, row_w[:, None])

    pos = comb_pos.reshape(T, K)
    g0 = _sc_row_gather(y_routed, pos[:, 0], T)
    g1 = _sc_row_gather(y_routed, pos[:, 1], T)

    NTS = T // B
    h_sh = pl.pallas_call(
        _shared_gate_up_body,
        grid=(NTS,),
        in_specs=[
            pl.BlockSpec((B, D), lambda i: (i, 0)),
            pl.BlockSpec((D, F), lambda i: (0, 0)),
            pl.BlockSpec((D, F), lambda i: (0, 0)),
        ],
        out_specs=pl.BlockSpec((B, F), lambda i: (i, 0)),
        out_shape=jax.ShapeDtypeStruct((T, F), jnp.float32),
    )(hidden_states, shared_w_gate, shared_w_up)

    out = pl.pallas_call(
        _shared_down_body,
        grid=(NTS,),
        in_specs=[
            pl.BlockSpec((B, F), lambda i: (i, 0)),
            pl.BlockSpec((F, D), lambda i: (0, 0)),
            pl.BlockSpec((B, D), lambda i: (i, 0)),
            pl.BlockSpec((B, D), lambda i: (i, 0)),
        ],
        out_specs=pl.BlockSpec((B, D), lambda i: (i, 0)),
        out_shape=jax.ShapeDtypeStruct((T, D), jnp.float32),
    )(h_sh, shared_w_down, g0, g1)
    return out
